# baseline (speedup 1.0000x reference)
import functools

import jax
import jax.numpy as jnp
from jax import lax
from jax.experimental import pallas as pl
from jax.experimental.pallas import tpu as pltpu
from jax.experimental.pallas import tpu_sc as plsc

N_NODES = 10000
N_EDGES = 320000
D = 128
N_GRAPHS = 64

NC = 2
NS = 16
NW = NC * NS
CHUNK = 128
CH0 = 112
CH1 = 48
CHW = CH0 + CH1
IDX_RING = 8
IDX_AHEAD = 6
E_PAD = NS * CHW * CHUNK
ROWS_PER_SUB = 624
TAIL_ROWS = N_NODES - NS * ROWS_PER_SUB
AGGR_ROWS = N_NODES + 16

NODE_BLK = 400
N_BLKS = N_NODES // NODE_BLK


def _sc_aggr_body(x_hbm, src_hbm, dst_hbm, zero_hbm, out_hbm,
                  src_idx, dst_idx, rows, sem, sem_i, sem_s, aggr):
    cid = lax.axis_index("c")
    sid = lax.axis_index("s")

    base = lax.select(cid == 0, 0, CH0)
    nch = lax.select(cid == 0, CH0, CH1)

    @pl.when(cid == 0)
    def _():
        pltpu.sync_copy(src_hbm.at[sid, pl.ds(0, CH0)], src_idx)

    @pl.when(cid == 1)
    def _():
        pltpu.sync_copy(src_hbm.at[sid, pl.ds(CH0, CH1)],
                        src_idx.at[pl.ds(0, CH1)])

    row0 = sid * ROWS_PER_SUB
    tail0 = NS * ROWS_PER_SUB

    @pl.when(cid == 0)
    def _():
        pltpu.sync_copy(x_hbm.at[pl.ds(row0, ROWS_PER_SUB)],
                        aggr.at[pl.ds(row0, ROWS_PER_SUB)])

        @pl.when(sid == 0)
        def _():
            pltpu.sync_copy(x_hbm.at[pl.ds(tail0, TAIL_ROWS)],
                            aggr.at[pl.ds(tail0, TAIL_ROWS)])

    @pl.when(cid == 1)
    def _():
        pltpu.sync_copy(zero_hbm.at[pl.ds(row0, ROWS_PER_SUB)],
                        aggr.at[pl.ds(row0, ROWS_PER_SUB)])

        @pl.when(sid == 0)
        def _():
            pltpu.sync_copy(zero_hbm.at[pl.ds(tail0, TAIL_ROWS)],
                            aggr.at[pl.ds(tail0, TAIL_ROWS)])

    plsc.subcore_barrier()

    pltpu.async_copy(x_hbm.at[src_idx.at[0]], rows.at[0], sem)
    for k in range(IDX_AHEAD):
        pltpu.async_copy(dst_hbm.at[sid, base + k], dst_idx.at[k], sem_i)

    def step(j, carry):
        buf = lax.rem(j, 2)
        nbuf = lax.rem(j + 1, 2)
        slot = lax.rem(j, IDX_RING)

        @pl.when(j >= 1)
        def _():
            pltpu.make_async_copy(
                rows.at[nbuf], aggr.at[dst_idx.at[lax.rem(j - 1, IDX_RING)]],
                sem_s).wait()

        @pl.when(j + 1 < nch)
        def _():
            pltpu.async_copy(x_hbm.at[src_idx.at[j + 1]], rows.at[nbuf], sem)

        @pl.when(j + IDX_AHEAD < nch)
        def _():
            pltpu.async_copy(dst_hbm.at[sid, base + j + IDX_AHEAD],
                             dst_idx.at[lax.rem(j + IDX_AHEAD, IDX_RING)],
                             sem_i)

        pltpu.make_async_copy(x_hbm.at[src_idx.at[j]], rows.at[buf], sem).wait()
        pltpu.make_async_copy(dst_hbm.at[sid, base + j], dst_idx.at[slot],
                              sem_i).wait()
        pltpu.async_copy(rows.at[buf], aggr.at[dst_idx.at[slot]], sem_s,
                         add=True)
        return carry

    lax.fori_loop(0, nch, step, 0)
    last = nch - 1
    pltpu.make_async_copy(rows.at[lax.rem(last, 2)],
                          aggr.at[dst_idx.at[lax.rem(last, IDX_RING)]],
                          sem_s).wait()

    plsc.subcore_barrier()

    pltpu.sync_copy(aggr.at[pl.ds(row0, ROWS_PER_SUB)],
                    out_hbm.at[cid, pl.ds(row0, ROWS_PER_SUB)])

    @pl.when(sid == 0)
    def _():
        pltpu.sync_copy(aggr.at[pl.ds(tail0, TAIL_ROWS)],
                        out_hbm.at[cid, pl.ds(tail0, TAIL_ROWS)])


def _sc_aggregate(x, srcp, dstp, zeros):
    mesh = plsc.VectorSubcoreMesh(core_axis_name="c", subcore_axis_name="s")
    fn = pl.kernel(
        _sc_aggr_body,
        mesh=mesh,
        out_type=jax.ShapeDtypeStruct((2, N_NODES, D), jnp.float32),
        scratch_types=[
            pltpu.VMEM((CH0, CHUNK), jnp.int32),
            pltpu.VMEM((IDX_RING, CHUNK), jnp.int32),
            pltpu.VMEM((2, CHUNK, D), jnp.float32),
            pltpu.SemaphoreType.DMA,
            pltpu.SemaphoreType.DMA,
            pltpu.SemaphoreType.DMA,
            pltpu.VMEM_SHARED((AGGR_ROWS, D), jnp.float32),
        ],
    )
    return fn(x, srcp, dstp, zeros)


def _tc_mlp_body(p_ref, batch_ref, wa_ref, ba_ref, wb_ref, bb_ref,
                 z_ref, g_ref, cacc):
    i = pl.program_id(0)
    h0 = p_ref[0] + p_ref[1]
    h = jnp.maximum(
        jnp.dot(h0, wa_ref[...], preferred_element_type=jnp.float32)
        + ba_ref[...], 0.0)
    z = jnp.maximum(
        jnp.dot(h, wb_ref[...], preferred_element_type=jnp.float32)
        + bb_ref[...], 0.0)
    z_ref[...] = z

    b = batch_ref[0, 0, :]
    onehot = (b[:, None]
              == lax.broadcasted_iota(jnp.int32, (NODE_BLK, N_GRAPHS), 1)
              ).astype(jnp.float32)
    gpart = lax.dot_general(onehot, z, (((0,), (0,)), ((), ())),
                            preferred_element_type=jnp.float32)
    cpart = lax.dot_general(onehot, jnp.ones_like(z), (((0,), (0,)), ((), ())),
                            preferred_element_type=jnp.float32)

    @pl.when(i == 0)
    def _():
        g_ref[...] = gpart
        cacc[...] = cpart

    @pl.when(i > 0)
    def _():
        g_ref[...] = g_ref[...] + gpart
        cacc[...] = cacc[...] + cpart

    @pl.when(i == N_BLKS - 1)
    def _():
        g_ref[...] = g_ref[...] / jnp.maximum(cacc[...], 1.0)


def _tc_mlp(p, batch3, wa, ba, wb, bb):
    return pl.pallas_call(
        _tc_mlp_body,
        grid=(N_BLKS,),
        in_specs=[
            pl.BlockSpec((2, NODE_BLK, D), lambda i: (0, i, 0)),
            pl.BlockSpec((1, 1, NODE_BLK), lambda i: (i, 0, 0)),
            pl.BlockSpec((D, D), lambda i: (0, 0)),
            pl.BlockSpec((1, D), lambda i: (0, 0)),
            pl.BlockSpec((D, D), lambda i: (0, 0)),
            pl.BlockSpec((1, D), lambda i: (0, 0)),
        ],
        out_specs=[
            pl.BlockSpec((NODE_BLK, D), lambda i: (i, 0)),
            pl.BlockSpec((N_GRAPHS, D), lambda i: (0, 0)),
        ],
        out_shape=[
            jax.ShapeDtypeStruct((N_NODES, D), jnp.float32),
            jax.ShapeDtypeStruct((N_GRAPHS, D), jnp.float32),
        ],
        scratch_shapes=[pltpu.VMEM((N_GRAPHS, D), jnp.float32)],
    )(p, batch3, wa, ba, wb, bb)


@jax.jit
def _run(x, edge_index, batch, W0a, b0a, W0b, b0b, W1a, b1a, W1b, b1b):
    src = edge_index[0]
    dst = edge_index[1]
    pad = E_PAD - N_EDGES
    n0 = NS * CH0 * CHUNK

    def layout(e, fill):
        ep = jnp.concatenate([e, jnp.full((pad,), fill, jnp.int32)])
        return jnp.concatenate(
            [ep[:n0].reshape(NS, CH0, CHUNK),
             ep[n0:].reshape(NS, CH1, CHUNK)], axis=1)

    srcp = layout(src, 0)
    dstp = layout(dst, N_NODES)
    zeros = jnp.zeros((N_NODES, D), jnp.float32)
    batch3 = batch.reshape(N_BLKS, 1, NODE_BLK)

    p = _sc_aggregate(x, srcp, dstp, zeros)
    z1, g1 = _tc_mlp(p, batch3, W0a, b0a.reshape(1, D), W0b, b0b.reshape(1, D))
    p2 = _sc_aggregate(z1, srcp, dstp, zeros)
    z2, g2 = _tc_mlp(p2, batch3, W1a, b1a.reshape(1, D), W1b, b1b.reshape(1, D))
    return z2, jnp.concatenate([g1, g2], axis=1)


def kernel(x, edge_index, edge_weights, batch,
           W0a, b0a, W0b, b0b, W1a, b1a, W1b, b1b):
    del edge_weights
    return _run(x, edge_index, batch, W0a, b0a, W0b, b0b, W1a, b1a, W1b, b1b)


# speedup vs baseline: 3.1613x; 3.1613x over previous
import functools

import jax
import jax.numpy as jnp
from jax import lax
from jax.experimental import pallas as pl
from jax.experimental.pallas import tpu as pltpu
from jax.experimental.pallas import tpu_sc as plsc

N_NODES = 10000
N_EDGES = 320000
D = 128
N_GRAPHS = 64

NC = 2
NS = 16
NW = NC * NS
CHUNK = 128
CHUNKS_PER_W = 79
IDX_RING = 8
IDX_AHEAD = 6
E_PAD = NW * CHUNKS_PER_W * CHUNK
ROWS_PER_SUB = 624
TAIL_ROWS = N_NODES - NS * ROWS_PER_SUB
DUMMY_ROWS = 128
AGGR_ROWS = N_NODES + DUMMY_ROWS

NODE_BLK = 400
N_BLKS = N_NODES // NODE_BLK


def _sc_aggr_body(x_hbm, src_hbm, dst_hbm, zero_hbm, out_hbm,
                  src_idx, dst_idx, rows, sem, sem_i, sem_s, aggr):
    cid = lax.axis_index("c")
    sid = lax.axis_index("s")
    wid = sid * NC + cid

    pltpu.sync_copy(src_hbm.at[wid], src_idx)

    row0 = sid * ROWS_PER_SUB
    tail0 = NS * ROWS_PER_SUB

    @pl.when(cid == 0)
    def _():
        pltpu.sync_copy(x_hbm.at[pl.ds(row0, ROWS_PER_SUB)],
                        aggr.at[pl.ds(row0, ROWS_PER_SUB)])

        @pl.when(sid == 0)
        def _():
            pltpu.sync_copy(x_hbm.at[pl.ds(tail0, TAIL_ROWS)],
                            aggr.at[pl.ds(tail0, TAIL_ROWS)])

    @pl.when(cid == 1)
    def _():
        pltpu.sync_copy(zero_hbm.at[pl.ds(row0, ROWS_PER_SUB)],
                        aggr.at[pl.ds(row0, ROWS_PER_SUB)])

        @pl.when(sid == 0)
        def _():
            pltpu.sync_copy(zero_hbm.at[pl.ds(tail0, TAIL_ROWS)],
                            aggr.at[pl.ds(tail0, TAIL_ROWS)])

    plsc.subcore_barrier()

    pltpu.async_copy(x_hbm.at[src_idx.at[0]], rows.at[0], sem)
    for k in range(IDX_AHEAD):
        pltpu.async_copy(dst_hbm.at[wid, k], dst_idx.at[k], sem_i)

    def step(j, carry):
        buf = lax.rem(j, 2)
        nbuf = lax.rem(j + 1, 2)
        slot = lax.rem(j, IDX_RING)

        @pl.when(j >= 1)
        def _():
            pltpu.make_async_copy(
                rows.at[nbuf], aggr.at[dst_idx.at[lax.rem(j - 1, IDX_RING)]],
                sem_s).wait()

        @pl.when(j + 1 < CHUNKS_PER_W)
        def _():
            pltpu.async_copy(x_hbm.at[src_idx.at[j + 1]], rows.at[nbuf], sem)

        @pl.when(j + IDX_AHEAD < CHUNKS_PER_W)
        def _():
            pltpu.async_copy(dst_hbm.at[wid, j + IDX_AHEAD],
                             dst_idx.at[lax.rem(j + IDX_AHEAD, IDX_RING)],
                             sem_i)

        pltpu.make_async_copy(x_hbm.at[src_idx.at[j]], rows.at[buf], sem).wait()
        pltpu.make_async_copy(dst_hbm.at[wid, j], dst_idx.at[slot],
                              sem_i).wait()
        pltpu.async_copy(rows.at[buf], aggr.at[dst_idx.at[slot]], sem_s,
                         add=True)
        return carry

    lax.fori_loop(0, CHUNKS_PER_W, step, 0)
    last = CHUNKS_PER_W - 1
    pltpu.make_async_copy(rows.at[lax.rem(last, 2)],
                          aggr.at[dst_idx.at[lax.rem(last, IDX_RING)]],
                          sem_s).wait()

    plsc.subcore_barrier()

    pltpu.sync_copy(aggr.at[pl.ds(row0, ROWS_PER_SUB)],
                    out_hbm.at[cid, pl.ds(row0, ROWS_PER_SUB)])

    @pl.when(sid == 0)
    def _():
        pltpu.sync_copy(aggr.at[pl.ds(tail0, TAIL_ROWS)],
                        out_hbm.at[cid, pl.ds(tail0, TAIL_ROWS)])


def _sc_aggregate(x, srcp, dstp, zeros):
    mesh = plsc.VectorSubcoreMesh(core_axis_name="c", subcore_axis_name="s")
    fn = pl.kernel(
        _sc_aggr_body,
        mesh=mesh,
        out_type=jax.ShapeDtypeStruct((2, N_NODES, D), jnp.float32),
        scratch_types=[
            pltpu.VMEM((CHUNKS_PER_W, CHUNK), jnp.int32),
            pltpu.VMEM((IDX_RING, CHUNK), jnp.int32),
            pltpu.VMEM((2, CHUNK, D), jnp.float32),
            pltpu.SemaphoreType.DMA,
            pltpu.SemaphoreType.DMA,
            pltpu.SemaphoreType.DMA,
            pltpu.VMEM_SHARED((AGGR_ROWS, D), jnp.float32),
        ],
    )
    return fn(x, srcp, dstp, zeros)


def _tc_mlp_body(p_ref, batch_ref, wa_ref, ba_ref, wb_ref, bb_ref,
                 z_ref, g_ref, cacc):
    i = pl.program_id(0)
    h0 = p_ref[0] + p_ref[1]
    h = jnp.maximum(
        jnp.dot(h0, wa_ref[...], preferred_element_type=jnp.float32)
        + ba_ref[...], 0.0)
    z = jnp.maximum(
        jnp.dot(h, wb_ref[...], preferred_element_type=jnp.float32)
        + bb_ref[...], 0.0)
    z_ref[...] = z

    b = batch_ref[0, 0, :]
    onehot = (b[:, None]
              == lax.broadcasted_iota(jnp.int32, (NODE_BLK, N_GRAPHS), 1)
              ).astype(jnp.float32)
    gpart = lax.dot_general(onehot, z, (((0,), (0,)), ((), ())),
                            preferred_element_type=jnp.float32)
    cpart = lax.dot_general(onehot, jnp.ones_like(z), (((0,), (0,)), ((), ())),
                            preferred_element_type=jnp.float32)

    @pl.when(i == 0)
    def _():
        g_ref[...] = gpart
        cacc[...] = cpart

    @pl.when(i > 0)
    def _():
        g_ref[...] = g_ref[...] + gpart
        cacc[...] = cacc[...] + cpart

    @pl.when(i == N_BLKS - 1)
    def _():
        g_ref[...] = g_ref[...] / jnp.maximum(cacc[...], 1.0)


def _tc_mlp(p, batch3, wa, ba, wb, bb):
    return pl.pallas_call(
        _tc_mlp_body,
        grid=(N_BLKS,),
        in_specs=[
            pl.BlockSpec((2, NODE_BLK, D), lambda i: (0, i, 0)),
            pl.BlockSpec((1, 1, NODE_BLK), lambda i: (i, 0, 0)),
            pl.BlockSpec((D, D), lambda i: (0, 0)),
            pl.BlockSpec((1, D), lambda i: (0, 0)),
            pl.BlockSpec((D, D), lambda i: (0, 0)),
            pl.BlockSpec((1, D), lambda i: (0, 0)),
        ],
        out_specs=[
            pl.BlockSpec((NODE_BLK, D), lambda i: (i, 0)),
            pl.BlockSpec((N_GRAPHS, D), lambda i: (0, 0)),
        ],
        out_shape=[
            jax.ShapeDtypeStruct((N_NODES, D), jnp.float32),
            jax.ShapeDtypeStruct((N_GRAPHS, D), jnp.float32),
        ],
        scratch_shapes=[pltpu.VMEM((N_GRAPHS, D), jnp.float32)],
    )(p, batch3, wa, ba, wb, bb)


@jax.jit
def _run(x, edge_index, batch, W0a, b0a, W0b, b0b, W1a, b1a, W1b, b1b):
    src = edge_index[0]
    dst = edge_index[1]
    pad = E_PAD - N_EDGES
    pad_iota = jnp.arange(pad, dtype=jnp.int32)
    srcp = jnp.concatenate([src, pad_iota % N_NODES]
                           ).reshape(NW, CHUNKS_PER_W, CHUNK)
    dstp = jnp.concatenate([dst, N_NODES + pad_iota % DUMMY_ROWS]
                           ).reshape(NW, CHUNKS_PER_W, CHUNK)
    zeros = jnp.zeros((N_NODES, D), jnp.float32)
    batch3 = batch.reshape(N_BLKS, 1, NODE_BLK)

    p = _sc_aggregate(x, srcp, dstp, zeros)
    z1, g1 = _tc_mlp(p, batch3, W0a, b0a.reshape(1, D), W0b, b0b.reshape(1, D))
    p2 = _sc_aggregate(z1, srcp, dstp, zeros)
    z2, g2 = _tc_mlp(p2, batch3, W1a, b1a.reshape(1, D), W1b, b1b.reshape(1, D))
    return z2, jnp.concatenate([g1, g2], axis=1)


def kernel(x, edge_index, edge_weights, batch,
           W0a, b0a, W0b, b0b, W1a, b1a, W1b, b1b):
    del edge_weights
    return _run(x, edge_index, batch, W0a, b0a, W0b, b0b, W1a, b1a, W1b, b1b)


# speedup vs baseline: 3.3888x; 1.0720x over previous
import functools

import jax
import jax.numpy as jnp
from jax import lax
from jax.experimental import pallas as pl
from jax.experimental.pallas import tpu as pltpu
from jax.experimental.pallas import tpu_sc as plsc

N_NODES = 10000
N_EDGES = 320000
D = 128
N_GRAPHS = 64

NC = 2
NS = 16
NW = NC * NS
CHUNK = 128
CHUNKS_PER_W = 79
IDX_RING = 8
IDX_AHEAD = 6
E_PAD = NW * CHUNKS_PER_W * CHUNK
ROWS_PER_SUB = 624
TAIL_ROWS = N_NODES - NS * ROWS_PER_SUB
DUMMY_ROWS = 128
AGGR_ROWS = N_NODES + DUMMY_ROWS

NODE_BLK = 1000
N_BLKS = N_NODES // NODE_BLK


def _sc_aggr_body(x_hbm, src_hbm, dst_hbm, zero_hbm, out_hbm,
                  src_idx, dst_idx, rows, sem, sem_i, sem_s, aggr):
    cid = lax.axis_index("c")
    sid = lax.axis_index("s")
    wid = sid * NC + cid

    pltpu.sync_copy(src_hbm.at[wid], src_idx)

    pltpu.async_copy(x_hbm.at[src_idx.at[0]], rows.at[0], sem)
    for k in range(IDX_AHEAD):
        pltpu.async_copy(dst_hbm.at[wid, k], dst_idx.at[k], sem_i)

    row0 = sid * ROWS_PER_SUB
    tail0 = NS * ROWS_PER_SUB

    @pl.when(cid == 0)
    def _():
        pltpu.sync_copy(x_hbm.at[pl.ds(row0, ROWS_PER_SUB)],
                        aggr.at[pl.ds(row0, ROWS_PER_SUB)])

        @pl.when(sid == 0)
        def _():
            pltpu.sync_copy(x_hbm.at[pl.ds(tail0, TAIL_ROWS)],
                            aggr.at[pl.ds(tail0, TAIL_ROWS)])

    @pl.when(cid == 1)
    def _():
        pltpu.sync_copy(zero_hbm.at[pl.ds(row0, ROWS_PER_SUB)],
                        aggr.at[pl.ds(row0, ROWS_PER_SUB)])

        @pl.when(sid == 0)
        def _():
            pltpu.sync_copy(zero_hbm.at[pl.ds(tail0, TAIL_ROWS)],
                            aggr.at[pl.ds(tail0, TAIL_ROWS)])

    plsc.subcore_barrier()

    def step(j, carry):
        buf = lax.rem(j, 2)
        nbuf = lax.rem(j + 1, 2)
        slot = lax.rem(j, IDX_RING)

        @pl.when(j >= 1)
        def _():
            pltpu.make_async_copy(
                rows.at[nbuf], aggr.at[dst_idx.at[lax.rem(j - 1, IDX_RING)]],
                sem_s).wait()

        @pl.when(j + 1 < CHUNKS_PER_W)
        def _():
            pltpu.async_copy(x_hbm.at[src_idx.at[j + 1]], rows.at[nbuf], sem)

        @pl.when(j + IDX_AHEAD < CHUNKS_PER_W)
        def _():
            pltpu.async_copy(dst_hbm.at[wid, j + IDX_AHEAD],
                             dst_idx.at[lax.rem(j + IDX_AHEAD, IDX_RING)],
                             sem_i)

        pltpu.make_async_copy(x_hbm.at[src_idx.at[j]], rows.at[buf], sem).wait()
        pltpu.make_async_copy(dst_hbm.at[wid, j], dst_idx.at[slot],
                              sem_i).wait()
        pltpu.async_copy(rows.at[buf], aggr.at[dst_idx.at[slot]], sem_s,
                         add=True)
        return carry

    lax.fori_loop(0, CHUNKS_PER_W, step, 0)
    last = CHUNKS_PER_W - 1
    pltpu.make_async_copy(rows.at[lax.rem(last, 2)],
                          aggr.at[dst_idx.at[lax.rem(last, IDX_RING)]],
                          sem_s).wait()

    plsc.subcore_barrier()

    pltpu.sync_copy(aggr.at[pl.ds(row0, ROWS_PER_SUB)],
                    out_hbm.at[cid, pl.ds(row0, ROWS_PER_SUB)])

    @pl.when(sid == 0)
    def _():
        pltpu.sync_copy(aggr.at[pl.ds(tail0, TAIL_ROWS)],
                        out_hbm.at[cid, pl.ds(tail0, TAIL_ROWS)])


def _sc_aggregate(x, srcp, dstp, zeros):
    mesh = plsc.VectorSubcoreMesh(core_axis_name="c", subcore_axis_name="s")
    fn = pl.kernel(
        _sc_aggr_body,
        mesh=mesh,
        out_type=jax.ShapeDtypeStruct((2, N_NODES, D), jnp.float32),
        scratch_types=[
            pltpu.VMEM((CHUNKS_PER_W, CHUNK), jnp.int32),
            pltpu.VMEM((IDX_RING, CHUNK), jnp.int32),
            pltpu.VMEM((2, CHUNK, D), jnp.float32),
            pltpu.SemaphoreType.DMA,
            pltpu.SemaphoreType.DMA,
            pltpu.SemaphoreType.DMA,
            pltpu.VMEM_SHARED((AGGR_ROWS, D), jnp.float32),
        ],
    )
    return fn(x, srcp, dstp, zeros)


def _tc_mlp_body(p_ref, batch_ref, wa_ref, ba_ref, wb_ref, bb_ref,
                 z_ref, g_ref, cacc):
    i = pl.program_id(0)
    h0 = p_ref[0] + p_ref[1]
    h = jnp.maximum(
        jnp.dot(h0, wa_ref[...], preferred_element_type=jnp.float32)
        + ba_ref[...], 0.0)
    z = jnp.maximum(
        jnp.dot(h, wb_ref[...], preferred_element_type=jnp.float32)
        + bb_ref[...], 0.0)
    z_ref[...] = z

    b = batch_ref[0, 0, :]
    onehot = (b[:, None]
              == lax.broadcasted_iota(jnp.int32, (NODE_BLK, N_GRAPHS), 1)
              ).astype(jnp.float32)
    gpart = lax.dot_general(onehot, z, (((0,), (0,)), ((), ())),
                            preferred_element_type=jnp.float32)
    cpart = lax.dot_general(onehot, jnp.ones_like(z), (((0,), (0,)), ((), ())),
                            preferred_element_type=jnp.float32)

    @pl.when(i == 0)
    def _():
        g_ref[...] = gpart
        cacc[...] = cpart

    @pl.when(i > 0)
    def _():
        g_ref[...] = g_ref[...] + gpart
        cacc[...] = cacc[...] + cpart

    @pl.when(i == N_BLKS - 1)
    def _():
        g_ref[...] = g_ref[...] / jnp.maximum(cacc[...], 1.0)


def _tc_mlp(p, batch3, wa, ba, wb, bb):
    return pl.pallas_call(
        _tc_mlp_body,
        grid=(N_BLKS,),
        in_specs=[
            pl.BlockSpec((2, NODE_BLK, D), lambda i: (0, i, 0)),
            pl.BlockSpec((1, 1, NODE_BLK), lambda i: (i, 0, 0)),
            pl.BlockSpec((D, D), lambda i: (0, 0)),
            pl.BlockSpec((1, D), lambda i: (0, 0)),
            pl.BlockSpec((D, D), lambda i: (0, 0)),
            pl.BlockSpec((1, D), lambda i: (0, 0)),
        ],
        out_specs=[
            pl.BlockSpec((NODE_BLK, D), lambda i: (i, 0)),
            pl.BlockSpec((N_GRAPHS, D), lambda i: (0, 0)),
        ],
        out_shape=[
            jax.ShapeDtypeStruct((N_NODES, D), jnp.float32),
            jax.ShapeDtypeStruct((N_GRAPHS, D), jnp.float32),
        ],
        scratch_shapes=[pltpu.VMEM((N_GRAPHS, D), jnp.float32)],
    )(p, batch3, wa, ba, wb, bb)


@jax.jit
def _run(x, edge_index, batch, W0a, b0a, W0b, b0b, W1a, b1a, W1b, b1b):
    src = edge_index[0]
    dst = edge_index[1]
    pad = E_PAD - N_EDGES
    pad_iota = jnp.arange(pad, dtype=jnp.int32)
    srcp = jnp.concatenate([src, pad_iota % N_NODES]
                           ).reshape(NW, CHUNKS_PER_W, CHUNK)
    dstp = jnp.concatenate([dst, N_NODES + pad_iota % DUMMY_ROWS]
                           ).reshape(NW, CHUNKS_PER_W, CHUNK)
    zeros = jnp.zeros((N_NODES, D), jnp.float32)
    batch3 = batch.reshape(N_BLKS, 1, NODE_BLK)

    p = _sc_aggregate(x, srcp, dstp, zeros)
    z1, g1 = _tc_mlp(p, batch3, W0a, b0a.reshape(1, D), W0b, b0b.reshape(1, D))
    p2 = _sc_aggregate(z1, srcp, dstp, zeros)
    z2, g2 = _tc_mlp(p2, batch3, W1a, b1a.reshape(1, D), W1b, b1b.reshape(1, D))
    return z2, jnp.concatenate([g1, g2], axis=1)


def kernel(x, edge_index, edge_weights, batch,
           W0a, b0a, W0b, b0b, W1a, b1a, W1b, b1b):
    del edge_weights
    return _run(x, edge_index, batch, W0a, b0a, W0b, b0b, W1a, b1a, W1b, b1b)


# speedup vs baseline: 3.4689x; 1.0236x over previous
import functools

import jax
import jax.numpy as jnp
from jax import lax
from jax.experimental import pallas as pl
from jax.experimental.pallas import tpu as pltpu
from jax.experimental.pallas import tpu_sc as plsc

N_NODES = 10000
N_EDGES = 320000
D = 128
N_GRAPHS = 64

NC = 2
NS = 16
NW = NC * NS
CHUNK = 128
CHUNKS_PER_W = 79
IDX_RING = 8
IDX_AHEAD = 6
E_PAD = NW * CHUNKS_PER_W * CHUNK
ROWS_PER_SUB = 624
TAIL_ROWS = N_NODES - NS * ROWS_PER_SUB
DUMMY_ROWS = 128
AGGR_ROWS = N_NODES + DUMMY_ROWS
ZBUF_ROWS = 16

NODE_BLK = 1000
N_BLKS = N_NODES // NODE_BLK


def _sc_aggr_body(x_hbm, src_hbm, dst_hbm, out_hbm,
                  src_idx, dst_idx, rows, zbuf, sem, sem_i, sem_s, sem_z,
                  aggr):
    cid = lax.axis_index("c")
    sid = lax.axis_index("s")
    wid = sid * NC + cid

    pltpu.sync_copy(src_hbm.at[wid], src_idx)

    pltpu.async_copy(x_hbm.at[src_idx.at[0]], rows.at[0], sem)
    for k in range(IDX_AHEAD):
        pltpu.async_copy(dst_hbm.at[wid, k], dst_idx.at[k], sem_i)

    row0 = sid * ROWS_PER_SUB
    tail0 = NS * ROWS_PER_SUB
    for r in range(ZBUF_ROWS):
        for c in range(D // 16):
            zbuf[r, pl.ds(c * 16, 16)] = jnp.zeros((16,), jnp.float32)
    for b in range(ROWS_PER_SUB // ZBUF_ROWS):
        pltpu.async_copy(zbuf, aggr.at[pl.ds(row0 + b * ZBUF_ROWS,
                                             ZBUF_ROWS)], sem_z)

    @pl.when(sid == 0)
    def _():
        pltpu.async_copy(zbuf, aggr.at[pl.ds(tail0, TAIL_ROWS)], sem_z)

    for b in range(ROWS_PER_SUB // ZBUF_ROWS):
        pltpu.make_async_copy(zbuf, aggr.at[pl.ds(row0 + b * ZBUF_ROWS,
                                                  ZBUF_ROWS)], sem_z).wait()

    @pl.when(sid == 0)
    def _():
        pltpu.make_async_copy(zbuf, aggr.at[pl.ds(tail0, TAIL_ROWS)],
                              sem_z).wait()

    plsc.subcore_barrier()

    def step(j, carry):
        buf = lax.rem(j, 2)
        nbuf = lax.rem(j + 1, 2)
        slot = lax.rem(j, IDX_RING)

        @pl.when(j >= 1)
        def _():
            pltpu.make_async_copy(
                rows.at[nbuf], aggr.at[dst_idx.at[lax.rem(j - 1, IDX_RING)]],
                sem_s).wait()

        @pl.when(j + 1 < CHUNKS_PER_W)
        def _():
            pltpu.async_copy(x_hbm.at[src_idx.at[j + 1]], rows.at[nbuf], sem)

        @pl.when(j + IDX_AHEAD < CHUNKS_PER_W)
        def _():
            pltpu.async_copy(dst_hbm.at[wid, j + IDX_AHEAD],
                             dst_idx.at[lax.rem(j + IDX_AHEAD, IDX_RING)],
                             sem_i)

        pltpu.make_async_copy(x_hbm.at[src_idx.at[j]], rows.at[buf], sem).wait()
        pltpu.make_async_copy(dst_hbm.at[wid, j], dst_idx.at[slot],
                              sem_i).wait()
        pltpu.async_copy(rows.at[buf], aggr.at[dst_idx.at[slot]], sem_s,
                         add=True)
        return carry

    lax.fori_loop(0, CHUNKS_PER_W, step, 0)
    last = CHUNKS_PER_W - 1
    pltpu.make_async_copy(rows.at[lax.rem(last, 2)],
                          aggr.at[dst_idx.at[lax.rem(last, IDX_RING)]],
                          sem_s).wait()

    plsc.subcore_barrier()

    pltpu.sync_copy(aggr.at[pl.ds(row0, ROWS_PER_SUB)],
                    out_hbm.at[cid, pl.ds(row0, ROWS_PER_SUB)])

    @pl.when(sid == 0)
    def _():
        pltpu.sync_copy(aggr.at[pl.ds(tail0, TAIL_ROWS)],
                        out_hbm.at[cid, pl.ds(tail0, TAIL_ROWS)])


def _sc_aggregate(x, srcp, dstp):
    mesh = plsc.VectorSubcoreMesh(core_axis_name="c", subcore_axis_name="s")
    fn = pl.kernel(
        _sc_aggr_body,
        mesh=mesh,
        out_type=jax.ShapeDtypeStruct((2, N_NODES, D), jnp.float32),
        scratch_types=[
            pltpu.VMEM((CHUNKS_PER_W, CHUNK), jnp.int32),
            pltpu.VMEM((IDX_RING, CHUNK), jnp.int32),
            pltpu.VMEM((2, CHUNK, D), jnp.float32),
            pltpu.VMEM((ZBUF_ROWS, D), jnp.float32),
            pltpu.SemaphoreType.DMA,
            pltpu.SemaphoreType.DMA,
            pltpu.SemaphoreType.DMA,
            pltpu.SemaphoreType.DMA,
            pltpu.VMEM_SHARED((AGGR_ROWS, D), jnp.float32),
        ],
    )
    return fn(x, srcp, dstp)


def _tc_mlp_body(x_ref, p_ref, batch_ref, wa_ref, ba_ref, wb_ref, bb_ref,
                 z_ref, g_ref, cacc):
    i = pl.program_id(0)
    h0 = x_ref[...] + p_ref[0] + p_ref[1]
    h = jnp.maximum(
        jnp.dot(h0, wa_ref[...], preferred_element_type=jnp.float32)
        + ba_ref[...], 0.0)
    z = jnp.maximum(
        jnp.dot(h, wb_ref[...], preferred_element_type=jnp.float32)
        + bb_ref[...], 0.0)
    z_ref[...] = z

    b = batch_ref[0, 0, :]
    onehot = (b[:, None]
              == lax.broadcasted_iota(jnp.int32, (NODE_BLK, N_GRAPHS), 1)
              ).astype(jnp.float32)
    gpart = lax.dot_general(onehot, z, (((0,), (0,)), ((), ())),
                            preferred_element_type=jnp.float32)
    cpart = lax.dot_general(onehot, jnp.ones_like(z), (((0,), (0,)), ((), ())),
                            preferred_element_type=jnp.float32)

    @pl.when(i == 0)
    def _():
        g_ref[...] = gpart
        cacc[...] = cpart

    @pl.when(i > 0)
    def _():
        g_ref[...] = g_ref[...] + gpart
        cacc[...] = cacc[...] + cpart

    @pl.when(i == N_BLKS - 1)
    def _():
        g_ref[...] = g_ref[...] / jnp.maximum(cacc[...], 1.0)


def _tc_mlp(x, p, batch3, wa, ba, wb, bb):
    return pl.pallas_call(
        _tc_mlp_body,
        grid=(N_BLKS,),
        in_specs=[
            pl.BlockSpec((NODE_BLK, D), lambda i: (i, 0)),
            pl.BlockSpec((2, NODE_BLK, D), lambda i: (0, i, 0)),
            pl.BlockSpec((1, 1, NODE_BLK), lambda i: (i, 0, 0)),
            pl.BlockSpec((D, D), lambda i: (0, 0)),
            pl.BlockSpec((1, D), lambda i: (0, 0)),
            pl.BlockSpec((D, D), lambda i: (0, 0)),
            pl.BlockSpec((1, D), lambda i: (0, 0)),
        ],
        out_specs=[
            pl.BlockSpec((NODE_BLK, D), lambda i: (i, 0)),
            pl.BlockSpec((N_GRAPHS, D), lambda i: (0, 0)),
        ],
        out_shape=[
            jax.ShapeDtypeStruct((N_NODES, D), jnp.float32),
            jax.ShapeDtypeStruct((N_GRAPHS, D), jnp.float32),
        ],
        scratch_shapes=[pltpu.VMEM((N_GRAPHS, D), jnp.float32)],
    )(x, p, batch3, wa, ba, wb, bb)


@jax.jit
def _run(x, edge_index, batch, W0a, b0a, W0b, b0b, W1a, b1a, W1b, b1b):
    src = edge_index[0]
    dst = edge_index[1]
    pad = E_PAD - N_EDGES
    pad_iota = jnp.arange(pad, dtype=jnp.int32)
    srcp = jnp.concatenate([src, pad_iota % N_NODES]
                           ).reshape(NW, CHUNKS_PER_W, CHUNK)
    dstp = jnp.concatenate([dst, N_NODES + pad_iota % DUMMY_ROWS]
                           ).reshape(NW, CHUNKS_PER_W, CHUNK)
    batch3 = batch.reshape(N_BLKS, 1, NODE_BLK)

    p = _sc_aggregate(x, srcp, dstp)
    z1, g1 = _tc_mlp(x, p, batch3,
                     W0a, b0a.reshape(1, D), W0b, b0b.reshape(1, D))
    p2 = _sc_aggregate(z1, srcp, dstp)
    z2, g2 = _tc_mlp(z1, p2, batch3,
                     W1a, b1a.reshape(1, D), W1b, b1b.reshape(1, D))
    return z2, jnp.concatenate([g1, g2], axis=1)


def kernel(x, edge_index, edge_weights, batch,
           W0a, b0a, W0b, b0b, W1a, b1a, W1b, b1b):
    del edge_weights
    return _run(x, edge_index, batch, W0a, b0a, W0b, b0b, W1a, b1a, W1b, b1b)


# speedup vs baseline: 3.5667x; 1.0282x over previous
import functools

import jax
import jax.numpy as jnp
from jax import lax
from jax.experimental import pallas as pl
from jax.experimental.pallas import tpu as pltpu
from jax.experimental.pallas import tpu_sc as plsc

N_NODES = 10000
N_EDGES = 320000
D = 128
N_GRAPHS = 64

NC = 2
NS = 16
NW = NC * NS
CHUNK = 128
CHUNKS_PER_W = 79
IDX_RING = 8
IDX_AHEAD = 6
E_PAD = NW * CHUNKS_PER_W * CHUNK
ROWS_PER_SUB = 624
TAIL_ROWS = N_NODES - NS * ROWS_PER_SUB
DUMMY_ROWS = 128
AGGR_ROWS = N_NODES + DUMMY_ROWS
ZBUF_ROWS = 16

NODE_BLK = 2000
N_BLKS = N_NODES // NODE_BLK


def _sc_aggr_body(x_hbm, src_hbm, dst_hbm, out_hbm,
                  src_idx, dst_idx, rows, zbuf, sem, sem_i, sem_s, sem_z,
                  aggr):
    cid = lax.axis_index("c")
    sid = lax.axis_index("s")
    wid = sid * NC + cid

    pltpu.sync_copy(src_hbm.at[wid], src_idx)

    pltpu.async_copy(x_hbm.at[src_idx.at[0]], rows.at[0], sem)
    for k in range(IDX_AHEAD):
        pltpu.async_copy(dst_hbm.at[wid, k], dst_idx.at[k], sem_i)

    row0 = sid * ROWS_PER_SUB
    tail0 = NS * ROWS_PER_SUB
    for r in range(ZBUF_ROWS):
        for c in range(D // 16):
            zbuf[r, pl.ds(c * 16, 16)] = jnp.zeros((16,), jnp.float32)
    for b in range(ROWS_PER_SUB // ZBUF_ROWS):
        pltpu.async_copy(zbuf, aggr.at[pl.ds(row0 + b * ZBUF_ROWS,
                                             ZBUF_ROWS)], sem_z)

    @pl.when(sid == 0)
    def _():
        pltpu.async_copy(zbuf, aggr.at[pl.ds(tail0, TAIL_ROWS)], sem_z)

    for b in range(ROWS_PER_SUB // ZBUF_ROWS):
        pltpu.make_async_copy(zbuf, aggr.at[pl.ds(row0 + b * ZBUF_ROWS,
                                                  ZBUF_ROWS)], sem_z).wait()

    @pl.when(sid == 0)
    def _():
        pltpu.make_async_copy(zbuf, aggr.at[pl.ds(tail0, TAIL_ROWS)],
                              sem_z).wait()

    plsc.subcore_barrier()

    def step(j, carry):
        buf = lax.rem(j, 2)
        nbuf = lax.rem(j + 1, 2)
        slot = lax.rem(j, IDX_RING)

        @pl.when(j >= 1)
        def _():
            pltpu.make_async_copy(
                rows.at[nbuf], aggr.at[dst_idx.at[lax.rem(j - 1, IDX_RING)]],
                sem_s).wait()

        @pl.when(j + 1 < CHUNKS_PER_W)
        def _():
            pltpu.async_copy(x_hbm.at[src_idx.at[j + 1]], rows.at[nbuf], sem)

        @pl.when(j + IDX_AHEAD < CHUNKS_PER_W)
        def _():
            pltpu.async_copy(dst_hbm.at[wid, j + IDX_AHEAD],
                             dst_idx.at[lax.rem(j + IDX_AHEAD, IDX_RING)],
                             sem_i)

        pltpu.make_async_copy(x_hbm.at[src_idx.at[j]], rows.at[buf], sem).wait()
        pltpu.make_async_copy(dst_hbm.at[wid, j], dst_idx.at[slot],
                              sem_i).wait()
        pltpu.async_copy(rows.at[buf], aggr.at[dst_idx.at[slot]], sem_s,
                         add=True)
        return carry

    lax.fori_loop(0, CHUNKS_PER_W, step, 0)
    last = CHUNKS_PER_W - 1
    pltpu.make_async_copy(rows.at[lax.rem(last, 2)],
                          aggr.at[dst_idx.at[lax.rem(last, IDX_RING)]],
                          sem_s).wait()

    plsc.subcore_barrier()

    pltpu.sync_copy(aggr.at[pl.ds(row0, ROWS_PER_SUB)],
                    out_hbm.at[cid, pl.ds(row0, ROWS_PER_SUB)])

    @pl.when(sid == 0)
    def _():
        pltpu.sync_copy(aggr.at[pl.ds(tail0, TAIL_ROWS)],
                        out_hbm.at[cid, pl.ds(tail0, TAIL_ROWS)])


def _sc_aggregate(x, srcp, dstp):
    mesh = plsc.VectorSubcoreMesh(core_axis_name="c", subcore_axis_name="s")
    fn = pl.kernel(
        _sc_aggr_body,
        mesh=mesh,
        out_type=jax.ShapeDtypeStruct((2, N_NODES, D), jnp.float32),
        scratch_types=[
            pltpu.VMEM((CHUNKS_PER_W, CHUNK), jnp.int32),
            pltpu.VMEM((IDX_RING, CHUNK), jnp.int32),
            pltpu.VMEM((2, CHUNK, D), jnp.float32),
            pltpu.VMEM((ZBUF_ROWS, D), jnp.float32),
            pltpu.SemaphoreType.DMA,
            pltpu.SemaphoreType.DMA,
            pltpu.SemaphoreType.DMA,
            pltpu.SemaphoreType.DMA,
            pltpu.VMEM_SHARED((AGGR_ROWS, D), jnp.float32),
        ],
    )
    return fn(x, srcp, dstp)


def _tc_mlp_body(x_ref, p_ref, batch_ref, wa_ref, ba_ref, wb_ref, bb_ref,
                 z_ref, g_ref, cacc):
    i = pl.program_id(0)
    h0 = x_ref[...] + p_ref[0] + p_ref[1]
    h = jnp.maximum(
        jnp.dot(h0, wa_ref[...], preferred_element_type=jnp.float32)
        + ba_ref[...], 0.0)
    z = jnp.maximum(
        jnp.dot(h, wb_ref[...], preferred_element_type=jnp.float32)
        + bb_ref[...], 0.0)
    z_ref[...] = z

    b = batch_ref[0, 0, :]
    onehot = (b[:, None]
              == lax.broadcasted_iota(jnp.int32, (NODE_BLK, N_GRAPHS), 1)
              ).astype(jnp.float32)
    gpart = lax.dot_general(onehot, z, (((0,), (0,)), ((), ())),
                            preferred_element_type=jnp.float32)
    cpart = lax.dot_general(onehot, jnp.ones_like(z), (((0,), (0,)), ((), ())),
                            preferred_element_type=jnp.float32)

    @pl.when(i == 0)
    def _():
        g_ref[...] = gpart
        cacc[...] = cpart

    @pl.when(i > 0)
    def _():
        g_ref[...] = g_ref[...] + gpart
        cacc[...] = cacc[...] + cpart

    @pl.when(i == N_BLKS - 1)
    def _():
        g_ref[...] = g_ref[...] / jnp.maximum(cacc[...], 1.0)


def _tc_mlp(x, p, batch3, wa, ba, wb, bb):
    return pl.pallas_call(
        _tc_mlp_body,
        grid=(N_BLKS,),
        in_specs=[
            pl.BlockSpec((NODE_BLK, D), lambda i: (i, 0)),
            pl.BlockSpec((2, NODE_BLK, D), lambda i: (0, i, 0)),
            pl.BlockSpec((1, 1, NODE_BLK), lambda i: (i, 0, 0)),
            pl.BlockSpec((D, D), lambda i: (0, 0)),
            pl.BlockSpec((1, D), lambda i: (0, 0)),
            pl.BlockSpec((D, D), lambda i: (0, 0)),
            pl.BlockSpec((1, D), lambda i: (0, 0)),
        ],
        out_specs=[
            pl.BlockSpec((NODE_BLK, D), lambda i: (i, 0)),
            pl.BlockSpec((N_GRAPHS, D), lambda i: (0, 0)),
        ],
        out_shape=[
            jax.ShapeDtypeStruct((N_NODES, D), jnp.float32),
            jax.ShapeDtypeStruct((N_GRAPHS, D), jnp.float32),
        ],
        scratch_shapes=[pltpu.VMEM((N_GRAPHS, D), jnp.float32)],
    )(x, p, batch3, wa, ba, wb, bb)


@jax.jit
def _run(x, edge_index, batch, W0a, b0a, W0b, b0b, W1a, b1a, W1b, b1b):
    src = edge_index[0]
    dst = edge_index[1]
    pad = E_PAD - N_EDGES
    pad_iota = jnp.arange(pad, dtype=jnp.int32)
    srcp = jnp.concatenate([src, pad_iota % N_NODES]
                           ).reshape(NW, CHUNKS_PER_W, CHUNK)
    dstp = jnp.concatenate([dst, N_NODES + pad_iota % DUMMY_ROWS]
                           ).reshape(NW, CHUNKS_PER_W, CHUNK)
    batch3 = batch.reshape(N_BLKS, 1, NODE_BLK)

    p = _sc_aggregate(x, srcp, dstp)
    z1, g1 = _tc_mlp(x, p, batch3,
                     W0a, b0a.reshape(1, D), W0b, b0b.reshape(1, D))
    p2 = _sc_aggregate(z1, srcp, dstp)
    z2, g2 = _tc_mlp(z1, p2, batch3,
                     W1a, b1a.reshape(1, D), W1b, b1b.reshape(1, D))
    return z2, jnp.concatenate([g1, g2], axis=1)


def kernel(x, edge_index, edge_weights, batch,
           W0a, b0a, W0b, b0b, W1a, b1a, W1b, b1b):
    del edge_weights
    return _run(x, edge_index, batch, W0a, b0a, W0b, b0b, W1a, b1a, W1b, b1b)


# speedup vs baseline: 3.8396x; 1.0765x over previous
import functools

import jax
import jax.numpy as jnp
from jax import lax
from jax.experimental import pallas as pl
from jax.experimental.pallas import tpu as pltpu
from jax.experimental.pallas import tpu_sc as plsc

N_NODES = 10000
N_EDGES = 320000
D = 128
N_GRAPHS = 64

NC = 2
NS = 16
NW = NC * NS
CHUNK = 80
CHUNKS_PER_W = 125
N_BUF = 3
IDX_RING = 8
IDX_AHEAD = 6
E_PAD = NW * CHUNKS_PER_W * CHUNK
ROWS_PER_SUB = 624
TAIL_ROWS = N_NODES - NS * ROWS_PER_SUB
DUMMY_ROWS = 128
AGGR_ROWS = N_NODES + DUMMY_ROWS
ZBUF_ROWS = 8

NODE_BLK = 2000
N_BLKS = N_NODES // NODE_BLK


def _sc_aggr_body(x_hbm, src_hbm, dst_hbm, out_hbm,
                  src_idx, dst_idx, rows, zbuf, sem, sem_i, sem_s, sem_z,
                  aggr):
    cid = lax.axis_index("c")
    sid = lax.axis_index("s")
    wid = sid * NC + cid

    pltpu.sync_copy(src_hbm.at[wid], src_idx)

    pltpu.async_copy(x_hbm.at[src_idx.at[0]], rows.at[0], sem)
    pltpu.async_copy(x_hbm.at[src_idx.at[1]], rows.at[1], sem)
    for k in range(IDX_AHEAD):
        pltpu.async_copy(dst_hbm.at[wid, k], dst_idx.at[k], sem_i)

    row0 = sid * ROWS_PER_SUB
    tail0 = NS * ROWS_PER_SUB
    for r in range(ZBUF_ROWS):
        for c in range(D // 16):
            zbuf[r, pl.ds(c * 16, 16)] = jnp.zeros((16,), jnp.float32)
    for b in range(ROWS_PER_SUB // ZBUF_ROWS):
        pltpu.async_copy(zbuf, aggr.at[pl.ds(row0 + b * ZBUF_ROWS,
                                             ZBUF_ROWS)], sem_z)

    @pl.when(sid == 0)
    def _():
        for b in range(TAIL_ROWS // ZBUF_ROWS):
            pltpu.async_copy(zbuf, aggr.at[pl.ds(tail0 + b * ZBUF_ROWS,
                                                 ZBUF_ROWS)], sem_z)

    for b in range(ROWS_PER_SUB // ZBUF_ROWS):
        pltpu.make_async_copy(zbuf, aggr.at[pl.ds(row0 + b * ZBUF_ROWS,
                                                  ZBUF_ROWS)], sem_z).wait()

    @pl.when(sid == 0)
    def _():
        for b in range(TAIL_ROWS // ZBUF_ROWS):
            pltpu.make_async_copy(zbuf, aggr.at[pl.ds(tail0 + b * ZBUF_ROWS,
                                                      ZBUF_ROWS)],
                                  sem_z).wait()

    plsc.subcore_barrier()

    def step(j, carry):
        buf = lax.rem(j, N_BUF)
        slot = lax.rem(j, IDX_RING)

        @pl.when(j >= 1)
        def _():
            pltpu.make_async_copy(
                rows.at[lax.rem(j - 1, N_BUF)],
                aggr.at[dst_idx.at[lax.rem(j - 1, IDX_RING)]],
                sem_s).wait()

        @pl.when(j + 2 < CHUNKS_PER_W)
        def _():
            pltpu.async_copy(x_hbm.at[src_idx.at[j + 2]],
                             rows.at[lax.rem(j + 2, N_BUF)], sem)

        @pl.when(j + IDX_AHEAD < CHUNKS_PER_W)
        def _():
            pltpu.async_copy(dst_hbm.at[wid, j + IDX_AHEAD],
                             dst_idx.at[lax.rem(j + IDX_AHEAD, IDX_RING)],
                             sem_i)

        pltpu.make_async_copy(x_hbm.at[src_idx.at[j]], rows.at[buf], sem).wait()
        pltpu.make_async_copy(dst_hbm.at[wid, j], dst_idx.at[slot],
                              sem_i).wait()
        pltpu.async_copy(rows.at[buf], aggr.at[dst_idx.at[slot]], sem_s,
                         add=True)
        return carry

    lax.fori_loop(0, CHUNKS_PER_W, step, 0)
    last = CHUNKS_PER_W - 1
    pltpu.make_async_copy(rows.at[lax.rem(last, N_BUF)],
                          aggr.at[dst_idx.at[lax.rem(last, IDX_RING)]],
                          sem_s).wait()

    plsc.subcore_barrier()

    pltpu.sync_copy(aggr.at[pl.ds(row0, ROWS_PER_SUB)],
                    out_hbm.at[cid, pl.ds(row0, ROWS_PER_SUB)])

    @pl.when(sid == 0)
    def _():
        pltpu.sync_copy(aggr.at[pl.ds(tail0, TAIL_ROWS)],
                        out_hbm.at[cid, pl.ds(tail0, TAIL_ROWS)])


def _sc_aggregate(x, srcp, dstp):
    mesh = plsc.VectorSubcoreMesh(core_axis_name="c", subcore_axis_name="s")
    fn = pl.kernel(
        _sc_aggr_body,
        mesh=mesh,
        out_type=jax.ShapeDtypeStruct((2, N_NODES, D), jnp.float32),
        scratch_types=[
            pltpu.VMEM((CHUNKS_PER_W, CHUNK), jnp.int32),
            pltpu.VMEM((IDX_RING, CHUNK), jnp.int32),
            pltpu.VMEM((N_BUF, CHUNK, D), jnp.float32),
            pltpu.VMEM((ZBUF_ROWS, D), jnp.float32),
            pltpu.SemaphoreType.DMA,
            pltpu.SemaphoreType.DMA,
            pltpu.SemaphoreType.DMA,
            pltpu.SemaphoreType.DMA,
            pltpu.VMEM_SHARED((AGGR_ROWS, D), jnp.float32),
        ],
    )
    return fn(x, srcp, dstp)


def _tc_mlp_body(x_ref, p_ref, batch_ref, wa_ref, ba_ref, wb_ref, bb_ref,
                 z_ref, g_ref, cacc):
    i = pl.program_id(0)
    h0 = x_ref[...] + p_ref[0] + p_ref[1]
    h = jnp.maximum(
        jnp.dot(h0, wa_ref[...], preferred_element_type=jnp.float32)
        + ba_ref[...], 0.0)
    z = jnp.maximum(
        jnp.dot(h, wb_ref[...], preferred_element_type=jnp.float32)
        + bb_ref[...], 0.0)
    z_ref[...] = z

    b = batch_ref[0, 0, :]
    onehot = (b[:, None]
              == lax.broadcasted_iota(jnp.int32, (NODE_BLK, N_GRAPHS), 1)
              ).astype(jnp.float32)
    gpart = lax.dot_general(onehot, z, (((0,), (0,)), ((), ())),
                            preferred_element_type=jnp.float32)
    cpart = lax.dot_general(onehot, jnp.ones_like(z), (((0,), (0,)), ((), ())),
                            preferred_element_type=jnp.float32)

    @pl.when(i == 0)
    def _():
        g_ref[...] = gpart
        cacc[...] = cpart

    @pl.when(i > 0)
    def _():
        g_ref[...] = g_ref[...] + gpart
        cacc[...] = cacc[...] + cpart

    @pl.when(i == N_BLKS - 1)
    def _():
        g_ref[...] = g_ref[...] / jnp.maximum(cacc[...], 1.0)


def _tc_mlp(x, p, batch3, wa, ba, wb, bb):
    return pl.pallas_call(
        _tc_mlp_body,
        grid=(N_BLKS,),
        in_specs=[
            pl.BlockSpec((NODE_BLK, D), lambda i: (i, 0)),
            pl.BlockSpec((2, NODE_BLK, D), lambda i: (0, i, 0)),
            pl.BlockSpec((1, 1, NODE_BLK), lambda i: (i, 0, 0)),
            pl.BlockSpec((D, D), lambda i: (0, 0)),
            pl.BlockSpec((1, D), lambda i: (0, 0)),
            pl.BlockSpec((D, D), lambda i: (0, 0)),
            pl.BlockSpec((1, D), lambda i: (0, 0)),
        ],
        out_specs=[
            pl.BlockSpec((NODE_BLK, D), lambda i: (i, 0)),
            pl.BlockSpec((N_GRAPHS, D), lambda i: (0, 0)),
        ],
        out_shape=[
            jax.ShapeDtypeStruct((N_NODES, D), jnp.float32),
            jax.ShapeDtypeStruct((N_GRAPHS, D), jnp.float32),
        ],
        scratch_shapes=[pltpu.VMEM((N_GRAPHS, D), jnp.float32)],
    )(x, p, batch3, wa, ba, wb, bb)


@jax.jit
def _run(x, edge_index, batch, W0a, b0a, W0b, b0b, W1a, b1a, W1b, b1b):
    src = edge_index[0]
    dst = edge_index[1]
    pad = E_PAD - N_EDGES
    if pad:
        pad_iota = jnp.arange(pad, dtype=jnp.int32)
        src = jnp.concatenate([src, pad_iota % N_NODES])
        dst = jnp.concatenate([dst, N_NODES + pad_iota % DUMMY_ROWS])
    srcp = src.reshape(NW, CHUNKS_PER_W, CHUNK)
    dstp = dst.reshape(NW, CHUNKS_PER_W, CHUNK)
    batch3 = batch.reshape(N_BLKS, 1, NODE_BLK)

    p = _sc_aggregate(x, srcp, dstp)
    z1, g1 = _tc_mlp(x, p, batch3,
                     W0a, b0a.reshape(1, D), W0b, b0b.reshape(1, D))
    p2 = _sc_aggregate(z1, srcp, dstp)
    z2, g2 = _tc_mlp(z1, p2, batch3,
                     W1a, b1a.reshape(1, D), W1b, b1b.reshape(1, D))
    return z2, jnp.concatenate([g1, g2], axis=1)


def kernel(x, edge_index, edge_weights, batch,
           W0a, b0a, W0b, b0b, W1a, b1a, W1b, b1b):
    del edge_weights
    return _run(x, edge_index, batch, W0a, b0a, W0b, b0b, W1a, b1a, W1b, b1b)


# speedup vs baseline: 3.9706x; 1.0341x over previous
import functools

import jax
import jax.numpy as jnp
from jax import lax
from jax.experimental import pallas as pl
from jax.experimental.pallas import tpu as pltpu
from jax.experimental.pallas import tpu_sc as plsc

N_NODES = 10000
N_EDGES = 320000
D = 128
N_GRAPHS = 64

NC = 2
NS = 16
NW = NC * NS
CHUNK = 72
CHUNKS_PER_W = 139
EDGES_PER_W = CHUNKS_PER_W * CHUNK
N_BUF = 4
IDX_RING = 8
IDX_AHEAD = 6
E_PAD = NW * EDGES_PER_W
ROWS_PER_SUB = 624
TAIL_ROWS = N_NODES - NS * ROWS_PER_SUB
DUMMY_ROWS = 16
AGGR_ROWS = N_NODES + DUMMY_ROWS
ZBUF_ROWS = 8

NODE_BLK = 2000
N_BLKS = N_NODES // NODE_BLK


def _sc_aggr_body(x_hbm, src_hbm, dst_hbm, out_hbm,
                  src_idx, dst_idx, rows, zbuf, sem, sem_i, sem_s, sem_z,
                  aggr):
    cid = lax.axis_index("c")
    sid = lax.axis_index("s")
    wid = sid * NC + cid

    pltpu.sync_copy(src_hbm.at[wid], src_idx)

    for k in range(N_BUF - 1):
        pltpu.async_copy(x_hbm.at[src_idx.at[pl.ds(k * CHUNK, CHUNK)]],
                         rows.at[k], sem)
    for k in range(IDX_AHEAD):
        pltpu.async_copy(dst_hbm.at[wid, k], dst_idx.at[k], sem_i)

    row0 = sid * ROWS_PER_SUB
    tail0 = NS * ROWS_PER_SUB
    for r in range(ZBUF_ROWS):
        for c in range(D // 16):
            zbuf[r, pl.ds(c * 16, 16)] = jnp.zeros((16,), jnp.float32)
    for b in range(ROWS_PER_SUB // ZBUF_ROWS):
        pltpu.async_copy(zbuf, aggr.at[pl.ds(row0 + b * ZBUF_ROWS,
                                             ZBUF_ROWS)], sem_z)

    @pl.when(sid == 0)
    def _():
        for b in range(TAIL_ROWS // ZBUF_ROWS):
            pltpu.async_copy(zbuf, aggr.at[pl.ds(tail0 + b * ZBUF_ROWS,
                                                 ZBUF_ROWS)], sem_z)

    for b in range(ROWS_PER_SUB // ZBUF_ROWS):
        pltpu.make_async_copy(zbuf, aggr.at[pl.ds(row0 + b * ZBUF_ROWS,
                                                  ZBUF_ROWS)], sem_z).wait()

    @pl.when(sid == 0)
    def _():
        for b in range(TAIL_ROWS // ZBUF_ROWS):
            pltpu.make_async_copy(zbuf, aggr.at[pl.ds(tail0 + b * ZBUF_ROWS,
                                                      ZBUF_ROWS)],
                                  sem_z).wait()

    plsc.subcore_barrier()

    def step(j, carry):
        buf = lax.rem(j, N_BUF)
        slot = lax.rem(j, IDX_RING)

        @pl.when(j >= 1)
        def _():
            pltpu.make_async_copy(
                rows.at[lax.rem(j - 1, N_BUF)],
                aggr.at[dst_idx.at[lax.rem(j - 1, IDX_RING)]],
                sem_s).wait()

        @pl.when(j + N_BUF - 1 < CHUNKS_PER_W)
        def _():
            pltpu.async_copy(
                x_hbm.at[src_idx.at[pl.ds((j + N_BUF - 1) * CHUNK, CHUNK)]],
                rows.at[lax.rem(j + N_BUF - 1, N_BUF)], sem)

        @pl.when(j + IDX_AHEAD < CHUNKS_PER_W)
        def _():
            pltpu.async_copy(dst_hbm.at[wid, j + IDX_AHEAD],
                             dst_idx.at[lax.rem(j + IDX_AHEAD, IDX_RING)],
                             sem_i)

        pltpu.make_async_copy(x_hbm.at[src_idx.at[pl.ds(j * CHUNK, CHUNK)]],
                              rows.at[buf], sem).wait()
        pltpu.make_async_copy(dst_hbm.at[wid, j], dst_idx.at[slot],
                              sem_i).wait()
        pltpu.async_copy(rows.at[buf], aggr.at[dst_idx.at[slot]], sem_s,
                         add=True)
        return carry

    lax.fori_loop(0, CHUNKS_PER_W, step, 0)
    last = CHUNKS_PER_W - 1
    pltpu.make_async_copy(rows.at[lax.rem(last, N_BUF)],
                          aggr.at[dst_idx.at[lax.rem(last, IDX_RING)]],
                          sem_s).wait()

    plsc.subcore_barrier()

    pltpu.sync_copy(aggr.at[pl.ds(row0, ROWS_PER_SUB)],
                    out_hbm.at[cid, pl.ds(row0, ROWS_PER_SUB)])

    @pl.when(sid == 0)
    def _():
        pltpu.sync_copy(aggr.at[pl.ds(tail0, TAIL_ROWS)],
                        out_hbm.at[cid, pl.ds(tail0, TAIL_ROWS)])


def _sc_aggregate(x, srcp, dstp):
    mesh = plsc.VectorSubcoreMesh(core_axis_name="c", subcore_axis_name="s")
    fn = pl.kernel(
        _sc_aggr_body,
        mesh=mesh,
        out_type=jax.ShapeDtypeStruct((2, N_NODES, D), jnp.float32),
        scratch_types=[
            pltpu.VMEM((EDGES_PER_W,), jnp.int32),
            pltpu.VMEM((IDX_RING, CHUNK), jnp.int32),
            pltpu.VMEM((N_BUF, CHUNK, D), jnp.float32),
            pltpu.VMEM((ZBUF_ROWS, D), jnp.float32),
            pltpu.SemaphoreType.DMA,
            pltpu.SemaphoreType.DMA,
            pltpu.SemaphoreType.DMA,
            pltpu.SemaphoreType.DMA,
            pltpu.VMEM_SHARED((AGGR_ROWS, D), jnp.float32),
        ],
    )
    return fn(x, srcp, dstp)


def _tc_mlp_body(x_ref, p_ref, batch_ref, wa_ref, ba_ref, wb_ref, bb_ref,
                 z_ref, g_ref, cacc):
    i = pl.program_id(0)
    h0 = x_ref[...] + p_ref[0] + p_ref[1]
    h = jnp.maximum(
        jnp.dot(h0, wa_ref[...], preferred_element_type=jnp.float32)
        + ba_ref[...], 0.0)
    z = jnp.maximum(
        jnp.dot(h, wb_ref[...], preferred_element_type=jnp.float32)
        + bb_ref[...], 0.0)
    z_ref[...] = z

    b = batch_ref[0, 0, :]
    onehot = (b[:, None]
              == lax.broadcasted_iota(jnp.int32, (NODE_BLK, N_GRAPHS), 1)
              ).astype(jnp.float32)
    gpart = lax.dot_general(onehot, z, (((0,), (0,)), ((), ())),
                            preferred_element_type=jnp.float32)
    cpart = lax.dot_general(onehot, jnp.ones_like(z), (((0,), (0,)), ((), ())),
                            preferred_element_type=jnp.float32)

    @pl.when(i == 0)
    def _():
        g_ref[...] = gpart
        cacc[...] = cpart

    @pl.when(i > 0)
    def _():
        g_ref[...] = g_ref[...] + gpart
        cacc[...] = cacc[...] + cpart

    @pl.when(i == N_BLKS - 1)
    def _():
        g_ref[...] = g_ref[...] / jnp.maximum(cacc[...], 1.0)


def _tc_mlp(x, p, batch3, wa, ba, wb, bb):
    return pl.pallas_call(
        _tc_mlp_body,
        grid=(N_BLKS,),
        in_specs=[
            pl.BlockSpec((NODE_BLK, D), lambda i: (i, 0)),
            pl.BlockSpec((2, NODE_BLK, D), lambda i: (0, i, 0)),
            pl.BlockSpec((1, 1, NODE_BLK), lambda i: (i, 0, 0)),
            pl.BlockSpec((D, D), lambda i: (0, 0)),
            pl.BlockSpec((1, D), lambda i: (0, 0)),
            pl.BlockSpec((D, D), lambda i: (0, 0)),
            pl.BlockSpec((1, D), lambda i: (0, 0)),
        ],
        out_specs=[
            pl.BlockSpec((NODE_BLK, D), lambda i: (i, 0)),
            pl.BlockSpec((N_GRAPHS, D), lambda i: (0, 0)),
        ],
        out_shape=[
            jax.ShapeDtypeStruct((N_NODES, D), jnp.float32),
            jax.ShapeDtypeStruct((N_GRAPHS, D), jnp.float32),
        ],
        scratch_shapes=[pltpu.VMEM((N_GRAPHS, D), jnp.float32)],
    )(x, p, batch3, wa, ba, wb, bb)


@jax.jit
def _run(x, edge_index, batch, W0a, b0a, W0b, b0b, W1a, b1a, W1b, b1b):
    src = edge_index[0]
    dst = edge_index[1]
    pad = E_PAD - N_EDGES
    if pad:
        pad_iota = jnp.arange(pad, dtype=jnp.int32)
        src = jnp.concatenate([src, pad_iota % N_NODES])
        dst = jnp.concatenate([dst, N_NODES + pad_iota % DUMMY_ROWS])
    srcp = src.reshape(NW, EDGES_PER_W)
    dstp = dst.reshape(NW, CHUNKS_PER_W, CHUNK)
    batch3 = batch.reshape(N_BLKS, 1, NODE_BLK)

    p = _sc_aggregate(x, srcp, dstp)
    z1, g1 = _tc_mlp(x, p, batch3,
                     W0a, b0a.reshape(1, D), W0b, b0b.reshape(1, D))
    p2 = _sc_aggregate(z1, srcp, dstp)
    z2, g2 = _tc_mlp(z1, p2, batch3,
                     W1a, b1a.reshape(1, D), W1b, b1b.reshape(1, D))
    return z2, jnp.concatenate([g1, g2], axis=1)


def kernel(x, edge_index, edge_weights, batch,
           W0a, b0a, W0b, b0b, W1a, b1a, W1b, b1b):
    del edge_weights
    return _run(x, edge_index, batch, W0a, b0a, W0b, b0b, W1a, b1a, W1b, b1b)
